# Initial kernel scaffold; baseline (speedup 1.0000x reference)
import jax
import jax.numpy as jnp
from jax.experimental import pallas as pl


def kernel(feature, xyz, Wq, bq, Wk, bk, Wv, bv, Wc, bc):
    raise NotImplementedError("write your pallas kernel here")


# speedup vs baseline: 11.6815x; 11.6815x over previous
import functools

import jax
import jax.numpy as jnp
from jax import lax
from jax.experimental import pallas as pl
from jax.experimental.pallas import tpu as pltpu
from jax.experimental.pallas import tpu_sc as plsc

f32 = jnp.float32
K1, K2, KN = 17, 8, 16


def _proj_call(Xt, APt, ASt, s0, B, N, CX, td):
    D3 = 3 * td

    def body(xt_ref, apt_ref, ast_ref, s0_ref, pkv_ref, pq_ref, s_ref):
        xt = xt_ref[0]
        t = jnp.dot(xt, apt_ref[...], preferred_element_type=f32)
        pkv_ref[0] = t[:, : 2 * td]
        pq_ref[0] = t[:, 2 * td :]
        s_ref[0] = jnp.dot(xt, ast_ref[...], preferred_element_type=f32) + s0_ref[...]

    return pl.pallas_call(
        body,
        grid=(B,),
        in_specs=[
            pl.BlockSpec((1, N, CX), lambda b: (b, 0, 0)),
            pl.BlockSpec((CX, 4 * td), lambda b: (0, 0)),
            pl.BlockSpec((CX, D3), lambda b: (0, 0)),
            pl.BlockSpec((1, D3), lambda b: (0, 0)),
        ],
        out_specs=[
            pl.BlockSpec((1, N, 2 * td), lambda b: (b, 0, 0)),
            pl.BlockSpec((1, N, 2 * td), lambda b: (b, 0, 0)),
            pl.BlockSpec((1, N, D3), lambda b: (b, 0, 0)),
        ],
        out_shape=[
            jax.ShapeDtypeStruct((B, N, 2 * td), f32),
            jax.ShapeDtypeStruct((B, N, 2 * td), f32),
            jax.ShapeDtypeStruct((B, N, D3), f32),
        ],
    )(Xt, APt, ASt, s0)


def _knn_call(ptsT, xyz, B, N, R):
    NT = N // R

    def body(pts_ref, xyz_ref, idx_ref, d2_ref):
        INF = jnp.float32(jnp.inf)
        BIG = jnp.int32(2**30)
        b = pl.program_id(0)
        pr = pts_ref[0]
        xz = xyz_ref[0]
        d2 = None
        for c in range(3):
            t = pr[:, c][:, None] - xz[c, :][None, :]
            t = t * t
            d2 = t if d2 is None else d2 + t
        d2_ref[...] = d2
        lane = lax.broadcasted_iota(jnp.int32, (R, N), 1)
        base = b * N
        for t in range(K1):
            v = d2_ref[...]
            m = jnp.min(v, axis=1, keepdims=True)
            cand = jnp.where(v <= m, lane, BIG)
            amin = jnp.min(cand, axis=1, keepdims=True)
            if t > 0:
                idx_ref[0, :, t - 1] = amin[:, 0] + base
            if t < K1 - 1:
                d2_ref[...] = jnp.where(lane == amin, INF, v)

    return pl.pallas_call(
        body,
        grid=(B, NT),
        in_specs=[
            pl.BlockSpec((1, R, 3), lambda b, i: (b, i, 0)),
            pl.BlockSpec((1, 3, N), lambda b, i: (b, 0, 0)),
        ],
        out_specs=pl.BlockSpec((1, R, KN), lambda b, i: (b, i, 0)),
        out_shape=jax.ShapeDtypeStruct((B, N, KN), jnp.int32),
        scratch_shapes=[pltpu.VMEM((R, N), f32)],
    )(ptsT, xyz)


def _gather_call(table_kv, table_q, idx_kv, idx_q, D):
    TOT_KV = idx_kv.shape[0]
    TOT_Q = idx_q.shape[0]
    NW = 32
    CH = 128
    n_kv = TOT_KV // (NW * CH)
    n_q = TOT_Q // (NW * CH)
    mesh = plsc.VectorSubcoreMesh(core_axis_name="c", subcore_axis_name="s")

    @functools.partial(
        pl.kernel,
        mesh=mesh,
        out_type=[
            jax.ShapeDtypeStruct((TOT_KV, D), f32),
            jax.ShapeDtypeStruct((TOT_Q, D), f32),
        ],
        scratch_types=[
            pltpu.VMEM((CH,), jnp.int32),
            pltpu.VMEM((CH, D), f32),
            pltpu.SemaphoreType.DMA,
        ],
    )
    def gather_kernel(tkv_hbm, tq_hbm, ikv_hbm, iq_hbm, okv_hbm, oq_hbm,
                      idx_v, rows_v, sem):
        wid = lax.axis_index("s") * 2 + lax.axis_index("c")

        def chunk_kv(ci, carry):
            base = wid * (n_kv * CH) + ci * CH
            pltpu.sync_copy(ikv_hbm.at[pl.ds(base, CH)], idx_v)
            pltpu.async_copy(tkv_hbm.at[idx_v], rows_v, sem).wait()
            pltpu.sync_copy(rows_v, okv_hbm.at[pl.ds(base, CH)])
            return carry

        def chunk_q(ci, carry):
            base = wid * (n_q * CH) + ci * CH
            pltpu.sync_copy(iq_hbm.at[pl.ds(base, CH)], idx_v)
            pltpu.async_copy(tq_hbm.at[idx_v], rows_v, sem).wait()
            pltpu.sync_copy(rows_v, oq_hbm.at[pl.ds(base, CH)])
            return carry

        lax.fori_loop(0, n_kv, chunk_kv, 0)
        lax.fori_loop(0, n_q, chunk_q, 0)

    return gather_kernel(table_kv, table_q, idx_kv, idx_q)


def _attn_call(Gkv, Gq, S, featT, WcT, bc2, B, N, C, td, Tn):
    NT = N // Tn
    D3 = 3 * td

    def body(gkv_ref, gq_ref, s_ref, ft_ref, wct_ref, bc_ref, o_ref):
        gkv = gkv_ref[0]
        gq = gq_ref[0]
        s = s_ref[0]
        q = gq[:, :, 0:td] + s[:, None, 0:td]
        k = gkv[:, :, 0:td] + s[:, None, td:2 * td]
        v = gkv[:, :, td:2 * td] + s[:, None, 2 * td:3 * td]
        ats = []
        for i in range(K2):
            qi = q[:, i, :]
            ats.append(jnp.sum(qi[:, None, :] * k, axis=-1))
        attn = jnp.stack(ats, axis=1)
        mx = jnp.max(attn, axis=2, keepdims=True)
        e = jnp.exp(attn - mx)
        w8 = e / jnp.sum(e, axis=2, keepdims=True)
        w = jnp.sum(w8, axis=1)
        res = jnp.sum(w[:, :, None] * v, axis=1)
        o_ref[0] = (
            jnp.dot(res, wct_ref[...], preferred_element_type=f32)
            + bc_ref[...]
            + ft_ref[0]
        )

    return pl.pallas_call(
        body,
        grid=(B, NT),
        in_specs=[
            pl.BlockSpec((1, Tn, KN, 2 * td), lambda b, i: (b, i, 0, 0)),
            pl.BlockSpec((1, Tn, K2, 2 * td), lambda b, i: (b, i, 0, 0)),
            pl.BlockSpec((1, Tn, D3), lambda b, i: (b, i, 0)),
            pl.BlockSpec((1, Tn, C), lambda b, i: (b, i, 0)),
            pl.BlockSpec((td, C), lambda b, i: (0, 0)),
            pl.BlockSpec((1, C), lambda b, i: (0, 0)),
        ],
        out_specs=pl.BlockSpec((1, Tn, C), lambda b, i: (b, i, 0)),
        out_shape=jax.ShapeDtypeStruct((B, N, C), f32),
    )(Gkv, Gq, S, featT, WcT, bc2)


def kernel(feature, xyz, Wq, bq, Wk, bk, Wv, bv, Wc, bc):
    B, C, N = feature.shape
    td = Wq.shape[0]
    CX = 3 + C
    D3 = 3 * td

    def split(W):
        Wa, Wb, Wcc, Wd = W[:, :3], W[:, 3:6], W[:, 6:6 + C], W[:, 6 + C:]
        A = jnp.concatenate([Wa, Wcc], axis=1)
        Sw = jnp.concatenate([Wb - Wa, Wd - Wcc], axis=1)
        return A, Sw

    Aq, Bq = split(Wq)
    Ak, Bk = split(Wk)
    Av, Bv = split(Wv)
    APt = jnp.concatenate([Ak, Av, Aq, Aq], axis=0).T
    ASt = jnp.concatenate([Bq, Bk, Bv], axis=0).T
    s0 = jnp.concatenate([bq, bk, bv])[None, :]

    X = jnp.concatenate([xyz, feature], axis=1)
    Xt = jnp.transpose(X, (0, 2, 1))
    ptsT = jnp.transpose(xyz, (0, 2, 1))
    featT = jnp.transpose(feature, (0, 2, 1))

    Pkv, Pq2, S = _proj_call(Xt, APt, ASt, s0, B, N, CX, td)
    idx = _knn_call(ptsT, xyz, B, N, 128)

    idx_kv = idx.reshape(B * N * KN)
    idx_q = idx[:, :, :K2].reshape(B * N * K2)
    Gkv, Gq = _gather_call(
        Pkv.reshape(B * N, 2 * td), Pq2.reshape(B * N, 2 * td),
        idx_kv, idx_q, 2 * td)

    O = _attn_call(
        Gkv.reshape(B, N, KN, 2 * td), Gq.reshape(B, N, K2, 2 * td),
        S, featT, Wc.T, bc[None, :], B, N, C, td, 128)
    return jnp.transpose(O, (0, 2, 1))


# speedup vs baseline: 11.9335x; 1.0216x over previous
import functools

import jax
import jax.numpy as jnp
from jax import lax
from jax.experimental import pallas as pl
from jax.experimental.pallas import tpu as pltpu
from jax.experimental.pallas import tpu_sc as plsc

f32 = jnp.float32
K1, K2, KN = 17, 8, 16


def _proj_call(Xt, APt, ASt, s0, B, N, CX, td):
    def body(xt_ref, apt_ref, ast_ref, s0_ref, pkv_ref, pq_ref, s_ref):
        xt = xt_ref[0]
        t = jnp.dot(xt, apt_ref[...], preferred_element_type=f32)
        pkv_ref[0] = t[:, : 2 * td]
        pq_ref[0] = t[:, 2 * td :]
        s_ref[0] = jnp.dot(xt, ast_ref[...], preferred_element_type=f32) + s0_ref[...]

    return pl.pallas_call(
        body,
        grid=(B,),
        in_specs=[
            pl.BlockSpec((1, N, CX), lambda b: (b, 0, 0)),
            pl.BlockSpec((CX, 4 * td), lambda b: (0, 0)),
            pl.BlockSpec((CX, 2 * td), lambda b: (0, 0)),
            pl.BlockSpec((1, 2 * td), lambda b: (0, 0)),
        ],
        out_specs=[
            pl.BlockSpec((1, N, 2 * td), lambda b: (b, 0, 0)),
            pl.BlockSpec((1, N, 2 * td), lambda b: (b, 0, 0)),
            pl.BlockSpec((1, N, 2 * td), lambda b: (b, 0, 0)),
        ],
        out_shape=[
            jax.ShapeDtypeStruct((B, N, 2 * td), f32),
            jax.ShapeDtypeStruct((B, N, 2 * td), f32),
            jax.ShapeDtypeStruct((B, N, 2 * td), f32),
        ],
    )(Xt, APt, ASt, s0)


def _knn_call(ptsT, xyz, N, R):
    NT = N // R

    def body(pts_ref, xyz_ref, idx_ref, d2_ref):
        INF = jnp.float32(jnp.inf)
        BIGF = jnp.float32(2**24)
        pr = pts_ref[0]
        xz = xyz_ref[0]
        d2 = None
        for c in range(3):
            t = pr[:, c][:, None] - xz[c, :][None, :]
            t = t * t
            d2 = t if d2 is None else d2 + t
        d2_ref[...] = d2
        lanef = lax.broadcasted_iota(jnp.int32, (R, N), 1).astype(f32)
        for t in range(K1):
            v = d2_ref[...]
            m = jnp.min(v, axis=1, keepdims=True)
            cand = jnp.where(v <= m, lanef, BIGF)
            aminf = jnp.min(cand, axis=1, keepdims=True)
            if t > 0:
                idx_ref[0, :, t - 1] = aminf[:, 0].astype(jnp.int32)
            if t < K1 - 1:
                d2_ref[...] = jnp.where(lanef == aminf, INF, v)

    return pl.pallas_call(
        body,
        grid=(1, NT),
        in_specs=[
            pl.BlockSpec((1, R, 3), lambda b, i: (b, i, 0)),
            pl.BlockSpec((1, 3, N), lambda b, i: (b, 0, 0)),
        ],
        out_specs=pl.BlockSpec((1, R, KN), lambda b, i: (b, i, 0)),
        out_shape=jax.ShapeDtypeStruct((1, N, KN), jnp.int32),
        scratch_shapes=[pltpu.VMEM((R, N), f32)],
    )(ptsT, xyz)


def _gather_call(table_kv, table_q, idx_kv, idx_q, D):
    TOT_KV = idx_kv.shape[0]
    TOT_Q = idx_q.shape[0]
    NW = 32
    CH = 128
    n_kv = TOT_KV // (NW * CH)
    n_q = TOT_Q // (NW * CH)
    mesh = plsc.VectorSubcoreMesh(core_axis_name="c", subcore_axis_name="s")

    @functools.partial(
        pl.kernel,
        mesh=mesh,
        out_type=[
            jax.ShapeDtypeStruct((TOT_KV, D), f32),
            jax.ShapeDtypeStruct((TOT_Q, D), f32),
        ],
        scratch_types=[
            pltpu.VMEM((CH,), jnp.int32),
            pltpu.VMEM((CH, D), f32),
            pltpu.SemaphoreType.DMA,
        ],
    )
    def gather_kernel(tkv_hbm, tq_hbm, ikv_hbm, iq_hbm, okv_hbm, oq_hbm,
                      idx_v, rows_v, sem):
        wid = lax.axis_index("s") * 2 + lax.axis_index("c")

        def chunk_kv(ci, carry):
            base = wid * (n_kv * CH) + ci * CH
            pltpu.sync_copy(ikv_hbm.at[pl.ds(base, CH)], idx_v)
            pltpu.async_copy(tkv_hbm.at[idx_v], rows_v, sem).wait()
            pltpu.sync_copy(rows_v, okv_hbm.at[pl.ds(base, CH)])
            return carry

        def chunk_q(ci, carry):
            base = wid * (n_q * CH) + ci * CH
            pltpu.sync_copy(iq_hbm.at[pl.ds(base, CH)], idx_v)
            pltpu.async_copy(tq_hbm.at[idx_v], rows_v, sem).wait()
            pltpu.sync_copy(rows_v, oq_hbm.at[pl.ds(base, CH)])
            return carry

        lax.fori_loop(0, n_kv, chunk_kv, 0)
        lax.fori_loop(0, n_q, chunk_q, 0)

    return gather_kernel(table_kv, table_q, idx_kv, idx_q)


def _attn_call(Gkv, Gq, S, featT, WcT, bc2, N, C, td, Tn):
    NT = N // Tn

    def body(gkv_ref, gq_ref, s_ref, ft_ref, wct_ref, bc_ref, o_ref):
        gkv = gkv_ref[0]
        gq = gq_ref[0]
        s = s_ref[0]
        q = gq[:, :, 0:td] + s[:, None, 0:td]
        pk = gkv[:, :, 0:td]
        v = gkv[:, :, td:2 * td] + s[:, None, td:2 * td]
        w = jnp.zeros((Tn, KN), f32)
        for i in range(K2):
            qi = q[:, i, :]
            a = jnp.sum(qi[:, None, :] * pk, axis=-1)
            a = a - jnp.max(a, axis=-1, keepdims=True)
            e = jnp.exp(a)
            w = w + e / jnp.sum(e, axis=-1, keepdims=True)
        res = jnp.sum(w[:, :, None] * v, axis=1)
        o_ref[0] = (
            jnp.dot(res, wct_ref[...], preferred_element_type=f32)
            + bc_ref[...]
            + ft_ref[0]
        )

    return pl.pallas_call(
        body,
        grid=(1, NT),
        in_specs=[
            pl.BlockSpec((1, Tn, KN, 2 * td), lambda b, i: (b, i, 0, 0)),
            pl.BlockSpec((1, Tn, K2, 2 * td), lambda b, i: (b, i, 0, 0)),
            pl.BlockSpec((1, Tn, 2 * td), lambda b, i: (b, i, 0)),
            pl.BlockSpec((1, Tn, C), lambda b, i: (b, i, 0)),
            pl.BlockSpec((td, C), lambda b, i: (0, 0)),
            pl.BlockSpec((1, C), lambda b, i: (0, 0)),
        ],
        out_specs=pl.BlockSpec((1, Tn, C), lambda b, i: (b, i, 0)),
        out_shape=jax.ShapeDtypeStruct((1, N, C), f32),
    )(Gkv, Gq, S, featT, WcT, bc2)


def kernel(feature, xyz, Wq, bq, Wk, bk, Wv, bv, Wc, bc):
    B, C, N = feature.shape
    td = Wq.shape[0]
    CX = 3 + C

    def split(W):
        Wa, Wb, Wcc, Wd = W[:, :3], W[:, 3:6], W[:, 6:6 + C], W[:, 6 + C:]
        A = jnp.concatenate([Wa, Wcc], axis=1)
        Sw = jnp.concatenate([Wb - Wa, Wd - Wcc], axis=1)
        return A, Sw

    Aq, Bq = split(Wq)
    Ak, _ = split(Wk)
    Av, Bv = split(Wv)
    APt = jnp.concatenate([Ak, Av, Aq, Aq], axis=0).T
    ASt = jnp.concatenate([Bq, Bv], axis=0).T
    s0 = jnp.concatenate([bq, bv])[None, :]

    X = jnp.concatenate([xyz, feature], axis=1)
    Xt = jnp.transpose(X, (0, 2, 1))
    ptsT = jnp.transpose(xyz, (0, 2, 1))
    featT = jnp.transpose(feature, (0, 2, 1))

    Pkv, Pq2, S = _proj_call(Xt, APt, ASt, s0, B, N, CX, td)

    WcT = Wc.T
    bc2 = bc[None, :]
    outs = []
    for b in range(B):
        idx = _knn_call(ptsT[b:b + 1], xyz[b:b + 1], N, 128)
        idx_kv = idx.reshape(N * KN)
        idx_q = idx[:, :, :K2].reshape(N * K2)
        Gkv, Gq = _gather_call(Pkv[b], Pq2[b], idx_kv, idx_q, 2 * td)
        O = _attn_call(
            Gkv.reshape(1, N, KN, 2 * td), Gq.reshape(1, N, K2, 2 * td),
            S[b:b + 1], featT[b:b + 1], WcT, bc2, N, C, td, 128)
        outs.append(O)
    return jnp.transpose(jnp.concatenate(outs, axis=0), (0, 2, 1))


# speedup vs baseline: 16.8778x; 1.4143x over previous
import functools

import jax
import jax.numpy as jnp
from jax import lax
from jax.experimental import pallas as pl
from jax.experimental.pallas import tpu as pltpu
from jax.experimental.pallas import tpu_sc as plsc

f32 = jnp.float32
K1, K2, KN = 17, 8, 16


def _proj_call(Xt, APt, ASt, s0, B, N, CX, td):
    def body(xt_ref, apt_ref, ast_ref, s0_ref, pkv_ref, pq_ref, s_ref):
        xt = xt_ref[0]
        t = jnp.dot(xt, apt_ref[...], preferred_element_type=f32)
        pkv_ref[0] = t[:, : 2 * td]
        pq_ref[0] = t[:, 2 * td :]
        s_ref[0] = jnp.dot(xt, ast_ref[...], preferred_element_type=f32) + s0_ref[...]

    return pl.pallas_call(
        body,
        grid=(B,),
        in_specs=[
            pl.BlockSpec((1, N, CX), lambda b: (b, 0, 0)),
            pl.BlockSpec((CX, 4 * td), lambda b: (0, 0)),
            pl.BlockSpec((CX, 2 * td), lambda b: (0, 0)),
            pl.BlockSpec((1, 2 * td), lambda b: (0, 0)),
        ],
        out_specs=[
            pl.BlockSpec((1, N, 2 * td), lambda b: (b, 0, 0)),
            pl.BlockSpec((1, N, 2 * td), lambda b: (b, 0, 0)),
            pl.BlockSpec((1, N, 2 * td), lambda b: (b, 0, 0)),
        ],
        out_shape=[
            jax.ShapeDtypeStruct((B, N, 2 * td), f32),
            jax.ShapeDtypeStruct((B, N, 2 * td), f32),
            jax.ShapeDtypeStruct((B, N, 2 * td), f32),
        ],
    )(Xt, APt, ASt, s0)


def _knn_call(ptsT, xyz, N, R):
    NT = N // R

    def body(pts_ref, xyz_ref, idx_ref, d2_ref):
        INF = jnp.float32(jnp.inf)
        BIGF = jnp.float32(2**24)
        pr = pts_ref[0]
        xz = xyz_ref[0]
        d2 = None
        for c in range(3):
            t = pr[:, c][:, None] - xz[c, :][None, :]
            t = t * t
            d2 = t if d2 is None else d2 + t
        d2_ref[...] = d2
        lanef = lax.broadcasted_iota(jnp.int32, (R, N), 1).astype(f32)
        for t in range(K1):
            v = d2_ref[...]
            m = jnp.min(v, axis=1, keepdims=True)
            cand = jnp.where(v <= m, lanef, BIGF)
            aminf = jnp.min(cand, axis=1, keepdims=True)
            if t > 0:
                idx_ref[0, :, t - 1] = aminf[:, 0].astype(jnp.int32)
            if t < K1 - 1:
                d2_ref[...] = jnp.where(lanef == aminf, INF, v)

    return pl.pallas_call(
        body,
        grid=(1, NT),
        in_specs=[
            pl.BlockSpec((1, R, 3), lambda b, i: (b, i, 0)),
            pl.BlockSpec((1, 3, N), lambda b, i: (b, 0, 0)),
        ],
        out_specs=pl.BlockSpec((1, R, KN), lambda b, i: (b, i, 0)),
        out_shape=jax.ShapeDtypeStruct((1, N, KN), jnp.int32),
        scratch_shapes=[pltpu.VMEM((R, N), f32)],
    )(ptsT, xyz)


def _gather_call(table_kv, table_q, idx_kv, idx_q, D):
    TOT_KV = idx_kv.shape[0]
    TOT_Q = idx_q.shape[0]
    NW = 32
    CH = 128
    n_kv = TOT_KV // (NW * CH)
    n_q = TOT_Q // (NW * CH)
    ikv2 = idx_kv.reshape(NW * n_kv, CH)
    iq2 = idx_q.reshape(NW * n_q, CH)
    mesh = plsc.VectorSubcoreMesh(core_axis_name="c", subcore_axis_name="s")

    @functools.partial(
        pl.kernel,
        mesh=mesh,
        out_type=[
            jax.ShapeDtypeStruct((TOT_KV, D), f32),
            jax.ShapeDtypeStruct((TOT_Q, D), f32),
        ],
        scratch_types=[
            pltpu.VMEM((n_kv, CH), jnp.int32),
            pltpu.VMEM((n_q, CH), jnp.int32),
            pltpu.VMEM((2, CH, D), f32),
            pltpu.SemaphoreType.DMA,
            pltpu.SemaphoreType.DMA,
            pltpu.SemaphoreType.DMA,
            pltpu.SemaphoreType.DMA,
        ],
    )
    def gather_kernel(tkv_hbm, tq_hbm, ikv_hbm, iq_hbm, okv_hbm, oq_hbm,
                      ikv_v, iq_v, rows2, sem_g0, sem_g1, sem_w0, sem_w1):
        sem_g = [sem_g0, sem_g1]
        sem_w = [sem_w0, sem_w1]
        wid = lax.axis_index("s") * 2 + lax.axis_index("c")
        pltpu.sync_copy(ikv_hbm.at[pl.ds(wid * n_kv, n_kv)], ikv_v)
        pltpu.sync_copy(iq_hbm.at[pl.ds(wid * n_q, n_q)], iq_v)

        chunks = ([(tkv_hbm, ikv_v, okv_hbm, c, wid * n_kv + c)
                   for c in range(n_kv)] +
                  [(tq_hbm, iq_v, oq_hbm, c, wid * n_q + c)
                   for c in range(n_q)])
        nc = len(chunks)

        def start_gather(j):
            tab, iv, _, c, _ = chunks[j]
            pltpu.async_copy(tab.at[iv.at[c]], rows2.at[j % 2], sem_g[j % 2])

        def wait_gather(j):
            tab, _, _, _, _ = chunks[j]
            pltpu.make_async_copy(
                tab.at[pl.ds(0, CH)], rows2.at[j % 2], sem_g[j % 2]).wait()

        def start_write(j):
            _, _, out, _, orow = chunks[j]
            pltpu.async_copy(
                rows2.at[j % 2], out.at[pl.ds(orow * CH, CH)], sem_w[j % 2])

        def wait_write(j):
            _, _, out, _, orow = chunks[j]
            pltpu.make_async_copy(
                rows2.at[j % 2], out.at[pl.ds(orow * CH, CH)], sem_w[j % 2]).wait()

        start_gather(0)
        for j in range(1, nc):
            if j >= 2:
                wait_write(j - 2)
            start_gather(j)
            wait_gather(j - 1)
            start_write(j - 1)
        wait_gather(nc - 1)
        start_write(nc - 1)
        wait_write(nc - 2)
        wait_write(nc - 1)

    return gather_kernel(table_kv, table_q, ikv2, iq2)


def _attn_call(Gkv, Gq, S, featT, WcT, bc2, N, C, td, Tn):
    NT = N // Tn

    def body(gkv_ref, gq_ref, s_ref, ft_ref, wct_ref, bc_ref, o_ref):
        gkv = gkv_ref[0]
        gq = gq_ref[0]
        s = s_ref[0]
        q = gq[:, :, 0:td] + s[:, None, 0:td]
        pk = gkv[:, :, 0:td]
        v = gkv[:, :, td:2 * td] + s[:, None, td:2 * td]
        ats = []
        for i in range(K2):
            qi = q[:, i, :]
            ats.append(jnp.sum(qi[:, None, :] * pk, axis=-1))
        attn = jnp.stack(ats, axis=1)
        mx = jnp.max(attn, axis=2, keepdims=True)
        e = jnp.exp(attn - mx)
        w8 = e / jnp.sum(e, axis=2, keepdims=True)
        w = jnp.sum(w8, axis=1)
        res = jnp.sum(w[:, :, None] * v, axis=1)
        o_ref[0] = (
            jnp.dot(res, wct_ref[...], preferred_element_type=f32)
            + bc_ref[...]
            + ft_ref[0]
        )

    return pl.pallas_call(
        body,
        grid=(1, NT),
        in_specs=[
            pl.BlockSpec((1, Tn, KN, 2 * td), lambda b, i: (b, i, 0, 0)),
            pl.BlockSpec((1, Tn, K2, 2 * td), lambda b, i: (b, i, 0, 0)),
            pl.BlockSpec((1, Tn, 2 * td), lambda b, i: (b, i, 0)),
            pl.BlockSpec((1, Tn, C), lambda b, i: (b, i, 0)),
            pl.BlockSpec((td, C), lambda b, i: (0, 0)),
            pl.BlockSpec((1, C), lambda b, i: (0, 0)),
        ],
        out_specs=pl.BlockSpec((1, Tn, C), lambda b, i: (b, i, 0)),
        out_shape=jax.ShapeDtypeStruct((1, N, C), f32),
    )(Gkv, Gq, S, featT, WcT, bc2)


def kernel(feature, xyz, Wq, bq, Wk, bk, Wv, bv, Wc, bc):
    B, C, N = feature.shape
    td = Wq.shape[0]
    CX = 3 + C

    def split(W):
        Wa, Wb, Wcc, Wd = W[:, :3], W[:, 3:6], W[:, 6:6 + C], W[:, 6 + C:]
        A = jnp.concatenate([Wa, Wcc], axis=1)
        Sw = jnp.concatenate([Wb - Wa, Wd - Wcc], axis=1)
        return A, Sw

    Aq, Bq = split(Wq)
    Ak, _ = split(Wk)
    Av, Bv = split(Wv)
    APt = jnp.concatenate([Ak, Av, Aq, Aq], axis=0).T
    ASt = jnp.concatenate([Bq, Bv], axis=0).T
    s0 = jnp.concatenate([bq, bv])[None, :]

    X = jnp.concatenate([xyz, feature], axis=1)
    Xt = jnp.transpose(X, (0, 2, 1))
    ptsT = jnp.transpose(xyz, (0, 2, 1))
    featT = jnp.transpose(feature, (0, 2, 1))

    Pkv, Pq2, S = _proj_call(Xt, APt, ASt, s0, B, N, CX, td)

    WcT = Wc.T
    bc2 = bc[None, :]
    outs = []
    for b in range(B):
        idx = _knn_call(ptsT[b:b + 1], xyz[b:b + 1], N, 128)
        idx_kv = idx.reshape(N * KN)
        idx_q = idx[:, :, :K2].reshape(N * K2)
        Gkv, Gq = _gather_call(Pkv[b], Pq2[b], idx_kv, idx_q, 2 * td)
        O = _attn_call(
            Gkv.reshape(1, N, KN, 2 * td), Gq.reshape(1, N, K2, 2 * td),
            S[b:b + 1], featT[b:b + 1], WcT, bc2, N, C, td, 128)
        outs.append(O)
    return jnp.transpose(jnp.concatenate(outs, axis=0), (0, 2, 1))


# speedup vs baseline: 16.8991x; 1.0013x over previous
import functools

import jax
import jax.numpy as jnp
from jax import lax
from jax.experimental import pallas as pl
from jax.experimental.pallas import tpu as pltpu
from jax.experimental.pallas import tpu_sc as plsc

f32 = jnp.float32
K1, K2, KN = 17, 8, 16


def _proj_call(Xt, APt, ASt, s0, B, N, CX, td):
    def body(xt_ref, apt_ref, ast_ref, s0_ref, pkv_ref, pq_ref, s_ref):
        xt = xt_ref[0]
        t = jnp.dot(xt, apt_ref[...], preferred_element_type=f32)
        pkv_ref[0] = t[:, : 2 * td]
        pq_ref[0] = t[:, 2 * td :]
        s_ref[0] = jnp.dot(xt, ast_ref[...], preferred_element_type=f32) + s0_ref[...]

    return pl.pallas_call(
        body,
        grid=(B,),
        in_specs=[
            pl.BlockSpec((1, N, CX), lambda b: (b, 0, 0)),
            pl.BlockSpec((CX, 4 * td), lambda b: (0, 0)),
            pl.BlockSpec((CX, 2 * td), lambda b: (0, 0)),
            pl.BlockSpec((1, 2 * td), lambda b: (0, 0)),
        ],
        out_specs=[
            pl.BlockSpec((1, N, 2 * td), lambda b: (b, 0, 0)),
            pl.BlockSpec((1, N, 2 * td), lambda b: (b, 0, 0)),
            pl.BlockSpec((1, N, 2 * td), lambda b: (b, 0, 0)),
        ],
        out_shape=[
            jax.ShapeDtypeStruct((B, N, 2 * td), f32),
            jax.ShapeDtypeStruct((B, N, 2 * td), f32),
            jax.ShapeDtypeStruct((B, N, 2 * td), f32),
        ],
    )(Xt, APt, ASt, s0)


def _knn_call(ptsT, xyz, N, R):
    NT = N // R

    def body(pts_ref, xyz_ref, idx_ref, d2_ref):
        INF = jnp.float32(jnp.inf)
        BIGF = jnp.float32(2**24)
        pr = pts_ref[0]
        xz = xyz_ref[0]
        d2 = None
        for c in range(3):
            t = pr[:, c][:, None] - xz[c, :][None, :]
            t = t * t
            d2 = t if d2 is None else d2 + t
        d2_ref[...] = d2
        lanef = lax.broadcasted_iota(jnp.int32, (R, N), 1).astype(f32)

        def extract(v):
            m = jnp.min(v, axis=1, keepdims=True)
            cand = jnp.where(v <= m, lanef, BIGF)
            aminf = jnp.min(cand, axis=1, keepdims=True)
            return aminf, jnp.where(lanef == aminf, INF, v)

        t = 0
        while t < K1:
            v = d2_ref[...]
            a1, v = extract(v)
            if t > 0:
                idx_ref[0, :, t - 1] = a1[:, 0].astype(jnp.int32)
            if t + 1 < K1:
                a2, v = extract(v)
                idx_ref[0, :, t] = a2[:, 0].astype(jnp.int32)
            if t + 2 < K1:
                d2_ref[...] = v
            t += 2

    return pl.pallas_call(
        body,
        grid=(1, NT),
        in_specs=[
            pl.BlockSpec((1, R, 3), lambda b, i: (b, i, 0)),
            pl.BlockSpec((1, 3, N), lambda b, i: (b, 0, 0)),
        ],
        out_specs=pl.BlockSpec((1, R, KN), lambda b, i: (b, i, 0)),
        out_shape=jax.ShapeDtypeStruct((1, N, KN), jnp.int32),
        scratch_shapes=[pltpu.VMEM((R, N), f32)],
    )(ptsT, xyz)


def _gather_call(table_kv, table_q, idx_kv, idx_q, D):
    TOT_KV = idx_kv.shape[0]
    TOT_Q = idx_q.shape[0]
    NW = 32
    CH = 128
    n_kv = TOT_KV // (NW * CH)
    n_q = TOT_Q // (NW * CH)
    ikv2 = idx_kv.reshape(NW * n_kv, CH)
    iq2 = idx_q.reshape(NW * n_q, CH)
    mesh = plsc.VectorSubcoreMesh(core_axis_name="c", subcore_axis_name="s")

    @functools.partial(
        pl.kernel,
        mesh=mesh,
        out_type=[
            jax.ShapeDtypeStruct((TOT_KV, D), f32),
            jax.ShapeDtypeStruct((TOT_Q, D), f32),
        ],
        scratch_types=[
            pltpu.VMEM((n_kv, CH), jnp.int32),
            pltpu.VMEM((n_q, CH), jnp.int32),
            pltpu.VMEM((2, CH, D), f32),
            pltpu.SemaphoreType.DMA,
            pltpu.SemaphoreType.DMA,
            pltpu.SemaphoreType.DMA,
            pltpu.SemaphoreType.DMA,
        ],
    )
    def gather_kernel(tkv_hbm, tq_hbm, ikv_hbm, iq_hbm, okv_hbm, oq_hbm,
                      ikv_v, iq_v, rows2, sem_g0, sem_g1, sem_w0, sem_w1):
        sem_g = [sem_g0, sem_g1]
        sem_w = [sem_w0, sem_w1]
        wid = lax.axis_index("s") * 2 + lax.axis_index("c")
        pltpu.sync_copy(ikv_hbm.at[pl.ds(wid * n_kv, n_kv)], ikv_v)
        pltpu.sync_copy(iq_hbm.at[pl.ds(wid * n_q, n_q)], iq_v)

        chunks = ([(tkv_hbm, ikv_v, okv_hbm, c, wid * n_kv + c)
                   for c in range(n_kv)] +
                  [(tq_hbm, iq_v, oq_hbm, c, wid * n_q + c)
                   for c in range(n_q)])
        nc = len(chunks)

        def start_gather(j):
            tab, iv, _, c, _ = chunks[j]
            pltpu.async_copy(tab.at[iv.at[c]], rows2.at[j % 2], sem_g[j % 2])

        def wait_gather(j):
            tab, _, _, _, _ = chunks[j]
            pltpu.make_async_copy(
                tab.at[pl.ds(0, CH)], rows2.at[j % 2], sem_g[j % 2]).wait()

        def start_write(j):
            _, _, out, _, orow = chunks[j]
            pltpu.async_copy(
                rows2.at[j % 2], out.at[pl.ds(orow * CH, CH)], sem_w[j % 2])

        def wait_write(j):
            _, _, out, _, orow = chunks[j]
            pltpu.make_async_copy(
                rows2.at[j % 2], out.at[pl.ds(orow * CH, CH)], sem_w[j % 2]).wait()

        start_gather(0)
        for j in range(1, nc):
            if j >= 2:
                wait_write(j - 2)
            start_gather(j)
            wait_gather(j - 1)
            start_write(j - 1)
        wait_gather(nc - 1)
        start_write(nc - 1)
        wait_write(nc - 2)
        wait_write(nc - 1)

    return gather_kernel(table_kv, table_q, ikv2, iq2)


def _attn_call(Gkv, Gq, S, featT, WcT, bc2, N, C, td, Tn):
    NT = N // Tn

    def body(gkv_ref, gq_ref, s_ref, ft_ref, wct_ref, bc_ref, o_ref):
        gkv = gkv_ref[0]
        gq = gq_ref[0]
        s = s_ref[0]
        q = gq[:, :, 0:td] + s[:, None, 0:td]
        pk = gkv[:, :, 0:td]
        v = gkv[:, :, td:2 * td] + s[:, None, td:2 * td]
        ats = []
        for i in range(K2):
            qi = q[:, i, :]
            ats.append(jnp.sum(qi[:, None, :] * pk, axis=-1))
        attn = jnp.stack(ats, axis=1)
        mx = jnp.max(attn, axis=2, keepdims=True)
        e = jnp.exp(attn - mx)
        w8 = e / jnp.sum(e, axis=2, keepdims=True)
        w = jnp.sum(w8, axis=1)
        res = jnp.sum(w[:, :, None] * v, axis=1)
        o_ref[0] = (
            jnp.dot(res, wct_ref[...], preferred_element_type=f32)
            + bc_ref[...]
            + ft_ref[0]
        )

    return pl.pallas_call(
        body,
        grid=(1, NT),
        in_specs=[
            pl.BlockSpec((1, Tn, KN, 2 * td), lambda b, i: (b, i, 0, 0)),
            pl.BlockSpec((1, Tn, K2, 2 * td), lambda b, i: (b, i, 0, 0)),
            pl.BlockSpec((1, Tn, 2 * td), lambda b, i: (b, i, 0)),
            pl.BlockSpec((1, Tn, C), lambda b, i: (b, i, 0)),
            pl.BlockSpec((td, C), lambda b, i: (0, 0)),
            pl.BlockSpec((1, C), lambda b, i: (0, 0)),
        ],
        out_specs=pl.BlockSpec((1, Tn, C), lambda b, i: (b, i, 0)),
        out_shape=jax.ShapeDtypeStruct((1, N, C), f32),
    )(Gkv, Gq, S, featT, WcT, bc2)


def kernel(feature, xyz, Wq, bq, Wk, bk, Wv, bv, Wc, bc):
    B, C, N = feature.shape
    td = Wq.shape[0]
    CX = 3 + C

    def split(W):
        Wa, Wb, Wcc, Wd = W[:, :3], W[:, 3:6], W[:, 6:6 + C], W[:, 6 + C:]
        A = jnp.concatenate([Wa, Wcc], axis=1)
        Sw = jnp.concatenate([Wb - Wa, Wd - Wcc], axis=1)
        return A, Sw

    Aq, Bq = split(Wq)
    Ak, _ = split(Wk)
    Av, Bv = split(Wv)
    APt = jnp.concatenate([Ak, Av, Aq, Aq], axis=0).T
    ASt = jnp.concatenate([Bq, Bv], axis=0).T
    s0 = jnp.concatenate([bq, bv])[None, :]

    X = jnp.concatenate([xyz, feature], axis=1)
    Xt = jnp.transpose(X, (0, 2, 1))
    ptsT = jnp.transpose(xyz, (0, 2, 1))
    featT = jnp.transpose(feature, (0, 2, 1))

    Pkv, Pq2, S = _proj_call(Xt, APt, ASt, s0, B, N, CX, td)

    WcT = Wc.T
    bc2 = bc[None, :]
    idxs = [_knn_call(ptsT[b:b + 1], xyz[b:b + 1], N, 128) for b in range(B)]
    gaths = []
    for b in range(B):
        idx = idxs[b]
        gaths.append(_gather_call(
            Pkv[b], Pq2[b], idx.reshape(N * KN),
            idx[:, :, :K2].reshape(N * K2), 2 * td))
    outs = []
    for b in range(B):
        Gkv, Gq = gaths[b]
        outs.append(_attn_call(
            Gkv.reshape(1, N, KN, 2 * td), Gq.reshape(1, N, K2, 2 * td),
            S[b:b + 1], featT[b:b + 1], WcT, bc2, N, C, td, 128))
    return jnp.transpose(jnp.concatenate(outs, axis=0), (0, 2, 1))
